# baseline (speedup 1.0000x reference)
import jax
import jax.numpy as jnp
from jax import lax
from jax.experimental import pallas as pl
from jax.experimental.pallas import tpu as pltpu
from jax.experimental.pallas import tpu_sc as plsc

N = 10000
E = 320000
G = 64
H = 256
EPS = 1e-5

NS = 16
N_PAD = 10240
SLICE = N_PAD // NS
EPT = 20480
E_PAD = NS * EPT
ITER = EPT // 16


def _fill(ref, n, value):
    v = jnp.full((16,), value, jnp.float32)

    @pl.loop(0, n // 16)
    def _(i):
        ref[pl.ds(i * 16, 16)] = v


def _rsqrt16(d):
    i = plsc.bitcast(d, jnp.int32)
    i = jnp.int32(0x5F3759DF) - lax.shift_right_logical(i, 1)
    y = plsc.bitcast(i, jnp.float32)
    half = d * 0.5
    for _ in range(3):
        y = y * (1.5 - half * y * y)
    return y


def _sc_kernel(src_hbm, dst_hbm, x_hbm, u_out, dinv_out, agg_parts, deg_parts,
               acc, u_v, src_v, dst_v, red, xbuf, dbuf):
    s = lax.axis_index("s")
    sl = pl.ds(s * SLICE, SLICE)
    _fill(acc, N_PAD, 0.0)
    pltpu.sync_copy(dst_hbm.at[pl.ds(s * EPT, EPT)], dst_v)
    pltpu.sync_copy(src_hbm.at[pl.ds(s * EPT, EPT)], src_v)
    ones = jnp.full((16,), 1.0, jnp.float32)

    @pl.loop(0, ITER, unroll=8)
    def _(k):
        idx = dst_v[pl.ds(k * 16, 16)]
        plsc.addupdate_scatter(acc, [idx], ones)

    pltpu.sync_copy(acc, deg_parts.at[s])
    plsc.subcore_barrier()

    pltpu.sync_copy(deg_parts.at[pl.ds(0, NS), sl], red)
    pltpu.sync_copy(x_hbm.at[sl], xbuf)

    @pl.loop(0, SLICE // 16)
    def _(i):
        ix = pl.ds(i * 16, 16)
        d = red[0, ix]
        for r in range(1, NS):
            d = d + red[r, ix]
        dinv = _rsqrt16(d + 1.0)
        dbuf[ix] = dinv
        xbuf[ix] = xbuf[ix] * dinv

    pltpu.sync_copy(xbuf, u_out.at[sl])
    pltpu.sync_copy(dbuf, dinv_out.at[sl])
    plsc.subcore_barrier()

    pltpu.sync_copy(u_out, u_v)
    _fill(acc, N_PAD, 0.0)

    @pl.loop(0, ITER, unroll=8)
    def _(k):
        six = src_v[pl.ds(k * 16, 16)]
        dix = dst_v[pl.ds(k * 16, 16)]
        vals = plsc.load_gather(u_v, [six])
        plsc.addupdate_scatter(acc, [dix], vals)

    pltpu.sync_copy(acc, agg_parts.at[s])


def _dot(a, b):
    return jnp.dot(a, b, precision=lax.Precision.HIGHEST,
                   preferred_element_type=jnp.float32)


def _head_body(aggp_ref, u_ref, dinv_ref, brow_ref, bcol_ref, W1c_ref, b1c_ref,
               gamma_ref, beta_ref, Wl1aT_ref, Wl1bT_ref, bl1c_ref, yfT_ref,
               Wl2T_ref, bl2c_ref, out_ref):
    agg = jnp.sum(aggp_ref[...], axis=0, keepdims=True)
    s_row = dinv_ref[...] * (agg + u_ref[...])
    h = jax.nn.gelu(W1c_ref[...] * s_row + b1c_ref[...])
    valid_row = (brow_ref[...] < G).astype(jnp.float32)
    hm = h * valid_row
    total = jnp.sum(hm, axis=1, keepdims=True)
    totalsq = jnp.sum(hm * hm, axis=1, keepdims=True)
    iota_g = lax.broadcasted_iota(jnp.int32, (N_PAD, G), 1)
    onehot_t = (bcol_ref[...] == iota_g).astype(jnp.float32)
    sums_t = _dot(h, onehot_t)
    counts = _dot(valid_row, onehot_t)
    mu = total * (1.0 / N)
    var = totalsq * (1.0 / N) - mu * mu
    pooled_t = sums_t / jnp.maximum(counts, 1.0)
    bn_t = (pooled_t - mu) * lax.rsqrt(var + EPS) * gamma_ref[...] + beta_ref[...]
    z1 = _dot(Wl1aT_ref[...], bn_t) + _dot(Wl1bT_ref[...], yfT_ref[...]) + bl1c_ref[...]
    g1 = jax.nn.gelu(z1)
    out_ref[...] = jax.nn.sigmoid(_dot(Wl2T_ref[...], g1) + bl2c_ref[...])


def kernel(x, edge_index, batch, y_feat, W1, b1, gamma, beta, Wl1, bl1, Wl2, bl2):
    f32 = jnp.float32
    pad = E_PAD - E
    src_f = jnp.concatenate([edge_index[0], jnp.zeros((pad,), jnp.int32)])
    dst_f = jnp.concatenate([edge_index[1], jnp.full((pad,), N, jnp.int32)])
    xp = jnp.pad(x[:, 0], (0, N_PAD - N))

    mesh = plsc.VectorSubcoreMesh(
        core_axis_name="c", subcore_axis_name="s", num_cores=1, num_subcores=NS
    )
    sc_call = pl.kernel(
        _sc_kernel,
        out_type=[
            jax.ShapeDtypeStruct((N_PAD,), f32),
            jax.ShapeDtypeStruct((N_PAD,), f32),
            jax.ShapeDtypeStruct((NS, N_PAD), f32),
            jax.ShapeDtypeStruct((NS, N_PAD), f32),
        ],
        mesh=mesh,
        compiler_params=pltpu.CompilerParams(needs_layout_passes=False),
        scratch_types=[
            pltpu.VMEM((N_PAD,), f32),
            pltpu.VMEM((N_PAD,), f32),
            pltpu.VMEM((EPT,), jnp.int32),
            pltpu.VMEM((EPT,), jnp.int32),
            pltpu.VMEM((NS, SLICE), f32),
            pltpu.VMEM((SLICE,), f32),
            pltpu.VMEM((SLICE,), f32),
        ],
    )
    u_flat, dinv_flat, agg_parts, _ = sc_call(src_f, dst_f, xp)

    batch_p = jnp.pad(batch, (0, N_PAD - N), constant_values=G)
    out_t = pl.pallas_call(
        _head_body,
        out_shape=jax.ShapeDtypeStruct((2, G), f32),
    )(
        agg_parts,
        u_flat.reshape(1, N_PAD),
        dinv_flat.reshape(1, N_PAD),
        batch_p.reshape(1, N_PAD),
        batch_p.reshape(N_PAD, 1),
        W1.reshape(H, 1),
        b1.reshape(H, 1),
        gamma.reshape(H, 1),
        beta.reshape(H, 1),
        Wl1[:H].T,
        Wl1[H:].T,
        bl1.reshape(-1, 1),
        y_feat.T,
        Wl2.T,
        bl2.reshape(-1, 1),
    )
    return out_t.T


# speedup vs baseline: 1.1821x; 1.1821x over previous
import jax
import jax.numpy as jnp
from jax import lax
from jax.experimental import pallas as pl
from jax.experimental.pallas import tpu as pltpu
from jax.experimental.pallas import tpu_sc as plsc

N = 10000
E = 320000
G = 64
H = 256
EPS = 1e-5

NS = 16
N_PAD = 10240
SLICE = N_PAD // NS
EPT = E // NS
ITER = EPT // 16


def _fill(ref, n, value):
    v = jnp.full((16,), value, jnp.float32)

    @pl.loop(0, n // 16)
    def _(i):
        ref[pl.ds(i * 16, 16)] = v


def _rsqrt16(d):
    i = plsc.bitcast(d, jnp.int32)
    i = jnp.int32(0x5F3759DF) - lax.shift_right_logical(i, 1)
    y = plsc.bitcast(i, jnp.float32)
    half = d * 0.5
    for _ in range(3):
        y = y * (1.5 - half * y * y)
    return y


def _sc_kernel(src_hbm, dst_hbm, x_hbm, u_out, dinv_out, agg_parts, deg_parts,
               acc, u_v, src_v, dst_v, red, xbuf, dbuf):
    s = lax.axis_index("s")
    sl = pl.ds(s * SLICE, SLICE)
    _fill(acc, N_PAD, 0.0)
    pltpu.sync_copy(dst_hbm.at[pl.ds(s * EPT, EPT)], dst_v)
    pltpu.sync_copy(src_hbm.at[pl.ds(s * EPT, EPT)], src_v)
    ones = jnp.full((16,), 1.0, jnp.float32)

    @pl.loop(0, ITER, unroll=16)
    def _(k):
        idx = dst_v[pl.ds(k * 16, 16)]
        plsc.addupdate_scatter(acc, [idx], ones)

    pltpu.sync_copy(acc, deg_parts.at[s])
    plsc.subcore_barrier()

    pltpu.sync_copy(deg_parts.at[pl.ds(0, NS), sl], red)
    pltpu.sync_copy(x_hbm.at[sl], xbuf)

    @pl.loop(0, SLICE // 16)
    def _(i):
        ix = pl.ds(i * 16, 16)
        d = red[0, ix]
        for r in range(1, NS):
            d = d + red[r, ix]
        dinv = _rsqrt16(d + 1.0)
        dbuf[ix] = dinv
        xbuf[ix] = xbuf[ix] * dinv

    pltpu.sync_copy(xbuf, u_out.at[sl])
    pltpu.sync_copy(dbuf, dinv_out.at[sl])
    plsc.subcore_barrier()

    pltpu.sync_copy(u_out, u_v)
    _fill(acc, N_PAD, 0.0)

    @pl.loop(0, ITER, unroll=16)
    def _(k):
        six = src_v[pl.ds(k * 16, 16)]
        dix = dst_v[pl.ds(k * 16, 16)]
        vals = plsc.load_gather(u_v, [six])
        plsc.addupdate_scatter(acc, [dix], vals)

    pltpu.sync_copy(acc, agg_parts.at[s])


def _dot(a, b):
    return jnp.dot(a, b, precision=lax.Precision.HIGHEST,
                   preferred_element_type=jnp.float32)


def _head_body(aggp_ref, u_ref, dinv_ref, brow_ref, bcol_ref, W1c_ref, b1c_ref,
               gamma_ref, beta_ref, Wl1aT_ref, Wl1bT_ref, bl1c_ref, yfT_ref,
               Wl2T_ref, bl2c_ref, out_ref):
    agg = jnp.sum(aggp_ref[...], axis=0, keepdims=True)
    s_row = dinv_ref[...] * (agg + u_ref[...])
    h = jax.nn.gelu(W1c_ref[...] * s_row + b1c_ref[...])
    valid_row = (brow_ref[...] < G).astype(jnp.float32)
    hm = h * valid_row
    total = jnp.sum(hm, axis=1, keepdims=True)
    totalsq = jnp.sum(hm * hm, axis=1, keepdims=True)
    iota_g = lax.broadcasted_iota(jnp.int32, (N_PAD, G), 1)
    onehot_t = (bcol_ref[...] == iota_g).astype(jnp.float32)
    sums_t = _dot(h, onehot_t)
    counts = jnp.sum(onehot_t, axis=0, keepdims=True)
    mu = total * (1.0 / N)
    var = totalsq * (1.0 / N) - mu * mu
    pooled_t = sums_t / jnp.maximum(counts, 1.0)
    bn_t = (pooled_t - mu) * lax.rsqrt(var + EPS) * gamma_ref[...] + beta_ref[...]
    z1 = _dot(Wl1aT_ref[...], bn_t) + _dot(Wl1bT_ref[...], yfT_ref[...]) + bl1c_ref[...]
    g1 = jax.nn.gelu(z1)
    out_ref[...] = jax.nn.sigmoid(_dot(Wl2T_ref[...], g1) + bl2c_ref[...])


def kernel(x, edge_index, batch, y_feat, W1, b1, gamma, beta, Wl1, bl1, Wl2, bl2):
    f32 = jnp.float32
    src_f = edge_index[0]
    dst_f = edge_index[1]
    xp = jnp.pad(x[:, 0], (0, N_PAD - N))

    mesh = plsc.VectorSubcoreMesh(
        core_axis_name="c", subcore_axis_name="s", num_cores=1, num_subcores=NS
    )
    sc_call = pl.kernel(
        _sc_kernel,
        out_type=[
            jax.ShapeDtypeStruct((N_PAD,), f32),
            jax.ShapeDtypeStruct((N_PAD,), f32),
            jax.ShapeDtypeStruct((NS, N_PAD), f32),
            jax.ShapeDtypeStruct((NS, N_PAD), f32),
        ],
        mesh=mesh,
        compiler_params=pltpu.CompilerParams(
            needs_layout_passes=False, disable_bounds_checks=True
        ),
        scratch_types=[
            pltpu.VMEM((N_PAD,), f32),
            pltpu.VMEM((N_PAD,), f32),
            pltpu.VMEM((EPT,), jnp.int32),
            pltpu.VMEM((EPT,), jnp.int32),
            pltpu.VMEM((NS, SLICE), f32),
            pltpu.VMEM((SLICE,), f32),
            pltpu.VMEM((SLICE,), f32),
        ],
    )
    u_flat, dinv_flat, agg_parts, _ = sc_call(src_f, dst_f, xp)

    batch_p = jnp.pad(batch, (0, N_PAD - N), constant_values=G)
    out_t = pl.pallas_call(
        _head_body,
        out_shape=jax.ShapeDtypeStruct((2, G), f32),
    )(
        agg_parts,
        u_flat.reshape(1, N_PAD),
        dinv_flat.reshape(1, N_PAD),
        batch_p.reshape(1, N_PAD),
        batch_p.reshape(N_PAD, 1),
        W1.reshape(H, 1),
        b1.reshape(H, 1),
        gamma.reshape(H, 1),
        beta.reshape(H, 1),
        Wl1[:H].T,
        Wl1[H:].T,
        bl1.reshape(-1, 1),
        y_feat.T,
        Wl2.T,
        bl2.reshape(-1, 1),
    )
    return out_t.T


# speedup vs baseline: 1.1971x; 1.0127x over previous
import jax
import jax.numpy as jnp
from jax import lax
from jax.experimental import pallas as pl
from jax.experimental.pallas import tpu as pltpu
from jax.experimental.pallas import tpu_sc as plsc

N = 10000
E = 320000
G = 64
H = 256
EPS = 1e-5

NS = 16
N_PAD = 10240
SLICE = N_PAD // NS
EPT = E // NS
ITER = EPT // 16


def _fill(ref, n, value):
    v = jnp.full((16,), value, jnp.float32)

    @pl.loop(0, n // 16)
    def _(i):
        ref[pl.ds(i * 16, 16)] = v


def _rsqrt16(d):
    i = plsc.bitcast(d, jnp.int32)
    i = jnp.int32(0x5F3759DF) - lax.shift_right_logical(i, 1)
    y = plsc.bitcast(i, jnp.float32)
    half = d * 0.5
    for _ in range(3):
        y = y * (1.5 - half * y * y)
    return y


def _sc_kernel(src_hbm, dst_hbm, x_hbm, u_out, dinv_out, agg_parts, deg_parts,
               acc, u_v, src_v, dst_v, red, xbuf, dbuf):
    s = lax.axis_index("s")
    sl = pl.ds(s * SLICE, SLICE)
    _fill(acc, N_PAD, 0.0)
    pltpu.sync_copy(dst_hbm.at[pl.ds(s * EPT, EPT)], dst_v)
    pltpu.sync_copy(src_hbm.at[pl.ds(s * EPT, EPT)], src_v)
    ones = jnp.full((16,), 1.0, jnp.float32)

    @pl.loop(0, ITER, unroll=32)
    def _(k):
        idx = dst_v[pl.ds(k * 16, 16)]
        plsc.addupdate_scatter(acc, [idx], ones)

    pltpu.sync_copy(acc, deg_parts.at[s])
    plsc.subcore_barrier()

    pltpu.sync_copy(deg_parts.at[pl.ds(0, NS), sl], red)
    pltpu.sync_copy(x_hbm.at[sl], xbuf)

    @pl.loop(0, SLICE // 16)
    def _(i):
        ix = pl.ds(i * 16, 16)
        d = red[0, ix]
        for r in range(1, NS):
            d = d + red[r, ix]
        dinv = _rsqrt16(d + 1.0)
        dbuf[ix] = dinv
        xbuf[ix] = xbuf[ix] * dinv

    pltpu.sync_copy(xbuf, u_out.at[sl])
    pltpu.sync_copy(dbuf, dinv_out.at[sl])
    plsc.subcore_barrier()

    pltpu.sync_copy(u_out, u_v)
    _fill(acc, N_PAD, 0.0)

    @pl.loop(0, ITER, unroll=32)
    def _(k):
        six = src_v[pl.ds(k * 16, 16)]
        dix = dst_v[pl.ds(k * 16, 16)]
        vals = plsc.load_gather(u_v, [six])
        plsc.addupdate_scatter(acc, [dix], vals)

    pltpu.sync_copy(acc, agg_parts.at[s])


def _dot(a, b):
    return jnp.dot(a, b, precision=lax.Precision.HIGHEST,
                   preferred_element_type=jnp.float32)


def _head_body(aggp_ref, u_ref, dinv_ref, brow_ref, bcol_ref, W1c_ref, b1c_ref,
               gamma_ref, beta_ref, Wl1aT_ref, Wl1bT_ref, bl1c_ref, yfT_ref,
               Wl2T_ref, bl2c_ref, out_ref):
    agg = jnp.sum(aggp_ref[...], axis=0, keepdims=True)
    s_row = dinv_ref[...] * (agg + u_ref[...])
    h = jax.nn.gelu(W1c_ref[...] * s_row + b1c_ref[...])
    valid_row = (brow_ref[...] < G).astype(jnp.float32)
    hm = h * valid_row
    total = jnp.sum(hm, axis=1, keepdims=True)
    totalsq = jnp.sum(hm * hm, axis=1, keepdims=True)
    iota_g = lax.broadcasted_iota(jnp.int32, (N_PAD, G), 1)
    bcol = bcol_ref[...].astype(jnp.int32)
    onehot_t = (bcol == iota_g).astype(jnp.float32)
    sums_t = _dot(h, onehot_t)
    counts = jnp.sum(onehot_t, axis=0, keepdims=True)
    mu = total * (1.0 / N)
    var = totalsq * (1.0 / N) - mu * mu
    pooled_t = sums_t / jnp.maximum(counts, 1.0)
    bn_t = (pooled_t - mu) * lax.rsqrt(var + EPS) * gamma_ref[...] + beta_ref[...]
    z1 = _dot(Wl1aT_ref[...], bn_t) + _dot(Wl1bT_ref[...], yfT_ref[...]) + bl1c_ref[...]
    g1 = jax.nn.gelu(z1)
    out_ref[...] = jax.nn.sigmoid(_dot(Wl2T_ref[...], g1) + bl2c_ref[...])


def kernel(x, edge_index, batch, y_feat, W1, b1, gamma, beta, Wl1, bl1, Wl2, bl2):
    f32 = jnp.float32
    src_f = edge_index[0]
    dst_f = edge_index[1]
    xp = jnp.pad(x[:, 0], (0, N_PAD - N))

    mesh = plsc.VectorSubcoreMesh(
        core_axis_name="c", subcore_axis_name="s", num_cores=1, num_subcores=NS
    )
    sc_call = pl.kernel(
        _sc_kernel,
        out_type=[
            jax.ShapeDtypeStruct((N_PAD,), f32),
            jax.ShapeDtypeStruct((N_PAD,), f32),
            jax.ShapeDtypeStruct((NS, N_PAD), f32),
            jax.ShapeDtypeStruct((NS, N_PAD), f32),
        ],
        mesh=mesh,
        compiler_params=pltpu.CompilerParams(
            needs_layout_passes=False, disable_bounds_checks=True
        ),
        scratch_types=[
            pltpu.VMEM((N_PAD,), f32),
            pltpu.VMEM((N_PAD,), f32),
            pltpu.VMEM((EPT,), jnp.int32),
            pltpu.VMEM((EPT,), jnp.int32),
            pltpu.VMEM((NS, SLICE), f32),
            pltpu.VMEM((SLICE,), f32),
            pltpu.VMEM((SLICE,), f32),
        ],
    )
    u_flat, dinv_flat, agg_parts, _ = sc_call(src_f, dst_f, xp)

    batch_p = jnp.pad(batch, (0, N_PAD - N), constant_values=G)
    out_t = pl.pallas_call(
        _head_body,
        out_shape=jax.ShapeDtypeStruct((2, G), f32),
    )(
        agg_parts,
        u_flat.reshape(1, N_PAD),
        dinv_flat.reshape(1, N_PAD),
        batch_p.reshape(1, N_PAD),
        batch_p.astype(jnp.int8).reshape(N_PAD, 1),
        W1.reshape(H, 1),
        b1.reshape(H, 1),
        gamma.reshape(H, 1),
        beta.reshape(H, 1),
        Wl1[:H].T,
        Wl1[H:].T,
        bl1.reshape(-1, 1),
        y_feat.T,
        Wl2.T,
        bl2.reshape(-1, 1),
    )
    return out_t.T
